# baseline (speedup 1.0000x reference)
import functools

import jax
import jax.numpy as jnp
from jax import lax
from jax.experimental import pallas as pl
from jax.experimental.pallas import tpu as pltpu
from jax.experimental.pallas import tpu_sc as plsc

EPS = 1e-5

NC = 2
NS = 16
NW = NC * NS

CH_IDX = 128
NBUF = 5

TBL_BLK = 8192


def _tc_encode_table(table_t, wt, b, gamma, beta, v, d, o):
    n_blk = (v + TBL_BLK - 1) // TBL_BLK
    vp = n_blk * TBL_BLK

    inv_o = 1.0 / o

    def body(x_ref, wt_ref, b_ref, g_ref, be_ref, o_ref):
        xt = x_ref[...]
        p = lax.dot_general(
            xt, wt_ref[...], (((0,), (0,)), ((), ())),
            preferred_element_type=jnp.float32,
        )
        p = p + b_ref[...]
        mean = jnp.sum(p, axis=1, keepdims=True) * inv_o
        s2 = jnp.sum(p * p, axis=1, keepdims=True) * inv_o
        var = s2 - mean * mean
        inv = lax.rsqrt(var + EPS)
        o_ref[...] = (p * inv - mean * inv) * g_ref[...] + be_ref[...]

    grid = (n_blk,)
    return pl.pallas_call(
        body,
        grid=grid,
        in_specs=[
            pl.BlockSpec((d, TBL_BLK), lambda i: (0, i)),
            pl.BlockSpec((d, o), lambda i: (0, 0)),
            pl.BlockSpec((1, o), lambda i: (0, 0)),
            pl.BlockSpec((1, o), lambda i: (0, 0)),
            pl.BlockSpec((1, o), lambda i: (0, 0)),
        ],
        out_specs=pl.BlockSpec((TBL_BLK, o), lambda i: (i, 0)),
        out_shape=jax.ShapeDtypeStruct((vp, o), jnp.float32),
        compiler_params=pltpu.CompilerParams(
            dimension_semantics=("arbitrary",),
        ),
    )(table_t, wt, b, gamma, beta)


def _sc_gather(ids3d, g_table, n, o):
    per_w = n // NW
    n_chunks = per_w // CH_IDX
    n_groups = n_chunks // NBUF
    mesh = plsc.VectorSubcoreMesh(
        core_axis_name="c", subcore_axis_name="s", num_cores=NC, num_subcores=NS
    )

    @functools.partial(
        pl.kernel,
        out_type=jax.ShapeDtypeStruct((n, o), jnp.float32),
        mesh=mesh,
        scratch_types=[
            pltpu.VMEM((n_chunks, CH_IDX), jnp.int32),
            pltpu.VMEM((NBUF, CH_IDX, o), jnp.float32),
            pltpu.SemaphoreType.DMA((NBUF,)),
            pltpu.SemaphoreType.DMA((NBUF,)),
        ],
        compiler_params=pltpu.CompilerParams(use_tc_tiling_on_sc=False),
    )
    def gather_kernel(ids_hbm, g_hbm, out_hbm, idx_v, rows_v, gsems, ssems):
        wid = lax.axis_index("s") * NC + lax.axis_index("c")
        base = wid * per_w
        pltpu.sync_copy(ids_hbm.at[wid], idx_v)

        def start_gather(j, buf):
            pltpu.async_copy(g_hbm.at[idx_v.at[j]], rows_v.at[buf], gsems.at[buf])

        def wait_gather(j, buf):
            pltpu.make_async_copy(
                g_hbm.at[idx_v.at[j]], rows_v.at[buf], gsems.at[buf]
            ).wait()

        def start_store(j, buf):
            pltpu.async_copy(
                rows_v.at[buf],
                out_hbm.at[pl.ds(base + j * CH_IDX, CH_IDX)],
                ssems.at[buf],
            )

        def wait_store(j, buf):
            pltpu.make_async_copy(
                rows_v.at[buf],
                out_hbm.at[pl.ds(base + j * CH_IDX, CH_IDX)],
                ssems.at[buf],
            ).wait()

        for buf in range(NBUF):
            start_gather(buf, buf)

        def group(g, carry):
            for buf in range(NBUF):
                j = g * NBUF + buf
                wait_gather(j, buf)
                start_store(j, buf)
            for buf in range(NBUF):
                j = g * NBUF + buf
                wait_store(j, buf)
                start_gather(j + NBUF, buf)
            return carry

        lax.fori_loop(0, n_groups - 1, group, 0, unroll=False)

        last = (n_groups - 1) * NBUF
        for buf in range(NBUF):
            wait_gather(last + buf, buf)
            start_store(last + buf, buf)
        for buf in range(NBUF):
            wait_store(last + buf, buf)

    return gather_kernel(ids3d, g_table)


@jax.jit
def kernel(item_ids, feat_matrix, W, b, gamma, beta):
    B, L = item_ids.shape
    v, d = feat_matrix.shape
    o = W.shape[0]
    n = B * L

    g_table = _tc_encode_table(
        feat_matrix.T,
        W.T,
        b.reshape(1, o),
        gamma.reshape(1, o),
        beta.reshape(1, o),
        v, d, o,
    )

    per_w = n // NW
    n_chunks = per_w // CH_IDX
    ids3d = item_ids.reshape(NW, n_chunks, CH_IDX)

    out = _sc_gather(ids3d, g_table, n, o)
    return out.reshape(B, L, o)


# speedup vs baseline: 1.2714x; 1.2714x over previous
import functools

import jax
import jax.numpy as jnp
from jax import lax
from jax.experimental import pallas as pl
from jax.experimental.pallas import tpu as pltpu
from jax.experimental.pallas import tpu_sc as plsc

EPS = 1e-5

NC = 2
NS = 16
NW = NC * NS

CH_IDX = 128
NBUF = 5

TBL_BLK = 8192


def _tc_encode_table(table_t, wt, b, gamma, beta, v, d, o):
    n_blk = (v + TBL_BLK - 1) // TBL_BLK
    vp = n_blk * TBL_BLK

    def body(x_ref, wt_ref, b_ref, g_ref, be_ref, o_ref):
        xt = x_ref[...]
        p2 = lax.dot_general(
            xt, wt_ref[...], (((0,), (0,)), ((), ())),
            preferred_element_type=jnp.float32,
        )
        p2 = p2 + b_ref[...]
        c = p2[:, :o] - p2[:, o:]
        var = jnp.mean(c * c, axis=1, keepdims=True)
        o_ref[...] = c * lax.rsqrt(var + EPS) * g_ref[...] + be_ref[...]

    wt_aug = jnp.concatenate(
        [wt, jnp.tile(jnp.mean(wt, axis=1, keepdims=True), (1, o))], axis=1
    )
    b_aug = jnp.concatenate(
        [b, jnp.full((1, o), jnp.mean(b), dtype=b.dtype)], axis=1
    )

    grid = (n_blk,)
    return pl.pallas_call(
        body,
        grid=grid,
        in_specs=[
            pl.BlockSpec((d, TBL_BLK), lambda i: (0, i)),
            pl.BlockSpec((d, 2 * o), lambda i: (0, 0)),
            pl.BlockSpec((1, 2 * o), lambda i: (0, 0)),
            pl.BlockSpec((1, o), lambda i: (0, 0)),
            pl.BlockSpec((1, o), lambda i: (0, 0)),
        ],
        out_specs=pl.BlockSpec((TBL_BLK, o), lambda i: (i, 0)),
        out_shape=jax.ShapeDtypeStruct((vp, o), jnp.float32),
        compiler_params=pltpu.CompilerParams(
            dimension_semantics=("arbitrary",),
        ),
    )(table_t, wt_aug, b_aug, gamma, beta)


def _sc_gather(ids3d, g_table, n, o):
    per_w = n // NW
    n_chunks = per_w // CH_IDX
    n_groups = n_chunks // NBUF
    mesh = plsc.VectorSubcoreMesh(
        core_axis_name="c", subcore_axis_name="s", num_cores=NC, num_subcores=NS
    )

    @functools.partial(
        pl.kernel,
        out_type=jax.ShapeDtypeStruct((n, o), jnp.float32),
        mesh=mesh,
        scratch_types=[
            pltpu.VMEM((n_chunks, CH_IDX), jnp.int32),
            pltpu.VMEM((NBUF, CH_IDX, o), jnp.float32),
            pltpu.SemaphoreType.DMA((NBUF,)),
            pltpu.SemaphoreType.DMA((NBUF,)),
        ],
        compiler_params=pltpu.CompilerParams(use_tc_tiling_on_sc=False),
    )
    def gather_kernel(ids_hbm, g_hbm, out_hbm, idx_v, rows_v, gsems, ssems):
        wid = lax.axis_index("s") * NC + lax.axis_index("c")
        base = wid * per_w
        pltpu.sync_copy(ids_hbm.at[wid], idx_v)

        def start_gather(j, buf):
            pltpu.async_copy(g_hbm.at[idx_v.at[j]], rows_v.at[buf], gsems.at[buf])

        def wait_gather(j, buf):
            pltpu.make_async_copy(
                g_hbm.at[idx_v.at[j]], rows_v.at[buf], gsems.at[buf]
            ).wait()

        def start_store(j, buf):
            pltpu.async_copy(
                rows_v.at[buf],
                out_hbm.at[pl.ds(base + j * CH_IDX, CH_IDX)],
                ssems.at[buf],
            )

        def wait_store(j, buf):
            pltpu.make_async_copy(
                rows_v.at[buf],
                out_hbm.at[pl.ds(base + j * CH_IDX, CH_IDX)],
                ssems.at[buf],
            ).wait()

        for buf in range(NBUF):
            start_gather(buf, buf)

        def group(g, carry):
            for buf in range(NBUF):
                j = g * NBUF + buf
                wait_gather(j, buf)
                start_store(j, buf)
            for buf in range(NBUF):
                j = g * NBUF + buf
                wait_store(j, buf)
                start_gather(j + NBUF, buf)
            return carry

        lax.fori_loop(0, n_groups - 1, group, 0, unroll=False)

        last = (n_groups - 1) * NBUF
        for buf in range(NBUF):
            wait_gather(last + buf, buf)
            start_store(last + buf, buf)
        for buf in range(NBUF):
            wait_store(last + buf, buf)

    return gather_kernel(ids3d, g_table)


@jax.jit
def kernel(item_ids, feat_matrix, W, b, gamma, beta):
    B, L = item_ids.shape
    v, d = feat_matrix.shape
    o = W.shape[0]
    n = B * L

    g_table = _tc_encode_table(
        feat_matrix.T,
        W.T,
        b.reshape(1, o),
        gamma.reshape(1, o),
        beta.reshape(1, o),
        v, d, o,
    )

    per_w = n // NW
    n_chunks = per_w // CH_IDX
    ids3d = item_ids.reshape(NW, n_chunks, CH_IDX)

    out = _sc_gather(ids3d, g_table, n, o)
    return out.reshape(B, L, o)


# speedup vs baseline: 1.3303x; 1.0463x over previous
import functools

import jax
import jax.numpy as jnp
from jax import lax
from jax.experimental import pallas as pl
from jax.experimental.pallas import tpu as pltpu
from jax.experimental.pallas import tpu_sc as plsc

EPS = 1e-5

NC = 2
NS = 16
NW = NC * NS

CH_IDX = 128
NBUF = 5

TBL_BLK = 16384


def _tc_encode_table(table_t, wt, b, gamma, beta, v, d, o):
    n_blk = (v + TBL_BLK - 1) // TBL_BLK
    vp = n_blk * TBL_BLK

    def body(x_ref, wt_ref, b_ref, g_ref, be_ref, o_ref):
        xt = x_ref[...]
        p2 = lax.dot_general(
            xt, wt_ref[...], (((0,), (0,)), ((), ())),
            preferred_element_type=jnp.float32,
        )
        p2 = p2 + b_ref[...]
        c = p2[:, :o] - p2[:, o:]
        var = jnp.mean(c * c, axis=1, keepdims=True)
        o_ref[...] = c * lax.rsqrt(var + EPS) * g_ref[...] + be_ref[...]

    wt_aug = jnp.concatenate(
        [wt, jnp.tile(jnp.mean(wt, axis=1, keepdims=True), (1, o))], axis=1
    )
    b_aug = jnp.concatenate(
        [b, jnp.full((1, o), jnp.mean(b), dtype=b.dtype)], axis=1
    )

    grid = (n_blk,)
    return pl.pallas_call(
        body,
        grid=grid,
        in_specs=[
            pl.BlockSpec((d, TBL_BLK), lambda i: (0, i)),
            pl.BlockSpec((d, 2 * o), lambda i: (0, 0)),
            pl.BlockSpec((1, 2 * o), lambda i: (0, 0)),
            pl.BlockSpec((1, o), lambda i: (0, 0)),
            pl.BlockSpec((1, o), lambda i: (0, 0)),
        ],
        out_specs=pl.BlockSpec((TBL_BLK, o), lambda i: (i, 0)),
        out_shape=jax.ShapeDtypeStruct((vp, o), jnp.float32),
        compiler_params=pltpu.CompilerParams(
            dimension_semantics=("arbitrary",),
        ),
    )(table_t, wt_aug, b_aug, gamma, beta)


def _sc_gather(ids3d, g_table, n, o):
    per_w = n // NW
    n_chunks = per_w // CH_IDX
    n_groups = n_chunks // NBUF
    mesh = plsc.VectorSubcoreMesh(
        core_axis_name="c", subcore_axis_name="s", num_cores=NC, num_subcores=NS
    )

    @functools.partial(
        pl.kernel,
        out_type=jax.ShapeDtypeStruct((n, o), jnp.float32),
        mesh=mesh,
        scratch_types=[
            pltpu.VMEM((n_chunks, CH_IDX), jnp.int32),
            pltpu.VMEM((NBUF, CH_IDX, o), jnp.float32),
            pltpu.SemaphoreType.DMA((NBUF,)),
            pltpu.SemaphoreType.DMA((NBUF,)),
        ],
        compiler_params=pltpu.CompilerParams(use_tc_tiling_on_sc=False),
    )
    def gather_kernel(ids_hbm, g_hbm, out_hbm, idx_v, rows_v, gsems, ssems):
        wid = lax.axis_index("s") * NC + lax.axis_index("c")
        base = wid * per_w
        pltpu.sync_copy(ids_hbm.at[wid], idx_v)

        def start_gather(j, buf):
            pltpu.async_copy(g_hbm.at[idx_v.at[j]], rows_v.at[buf], gsems.at[buf])

        def wait_gather(j, buf):
            pltpu.make_async_copy(
                g_hbm.at[idx_v.at[j]], rows_v.at[buf], gsems.at[buf]
            ).wait()

        def start_store(j, buf):
            pltpu.async_copy(
                rows_v.at[buf],
                out_hbm.at[pl.ds(base + j * CH_IDX, CH_IDX)],
                ssems.at[buf],
            )

        def wait_store(j, buf):
            pltpu.make_async_copy(
                rows_v.at[buf],
                out_hbm.at[pl.ds(base + j * CH_IDX, CH_IDX)],
                ssems.at[buf],
            ).wait()

        for buf in range(NBUF):
            start_gather(buf, buf)

        def group(g, carry):
            for buf in range(NBUF):
                j = g * NBUF + buf
                wait_gather(j, buf)
                start_store(j, buf)
            for buf in range(NBUF):
                j = g * NBUF + buf
                wait_store(j, buf)
                start_gather(j + NBUF, buf)
            return carry

        lax.fori_loop(0, n_groups - 1, group, 0, unroll=False)

        last = (n_groups - 1) * NBUF
        for buf in range(NBUF):
            wait_gather(last + buf, buf)
            start_store(last + buf, buf)
        for buf in range(NBUF):
            wait_store(last + buf, buf)

    return gather_kernel(ids3d, g_table)


@jax.jit
def kernel(item_ids, feat_matrix, W, b, gamma, beta):
    B, L = item_ids.shape
    v, d = feat_matrix.shape
    o = W.shape[0]
    n = B * L

    g_table = _tc_encode_table(
        feat_matrix.T,
        W.T,
        b.reshape(1, o),
        gamma.reshape(1, o),
        beta.reshape(1, o),
        v, d, o,
    )

    per_w = n // NW
    n_chunks = per_w // CH_IDX
    ids3d = item_ids.reshape(NW, n_chunks, CH_IDX)

    out = _sc_gather(ids3d, g_table, n, o)
    return out.reshape(B, L, o)


# speedup vs baseline: 1.3566x; 1.0197x over previous
import functools

import jax
import jax.numpy as jnp
from jax import lax
from jax.experimental import pallas as pl
from jax.experimental.pallas import tpu as pltpu
from jax.experimental.pallas import tpu_sc as plsc

EPS = 1e-5

NC = 2
NS = 16
NW = NC * NS

CH_IDX = 128
NBUF = 5

TBL_BLK = 24576


def _tc_encode_table(table_t, wt, b, gamma, beta, v, d, o):
    n_blk = (v + TBL_BLK - 1) // TBL_BLK
    vp = n_blk * TBL_BLK

    def body(x_ref, wt_ref, b_ref, g_ref, be_ref, o_ref):
        xt = x_ref[...]
        p2 = lax.dot_general(
            xt, wt_ref[...], (((0,), (0,)), ((), ())),
            preferred_element_type=jnp.float32,
        )
        p2 = p2 + b_ref[...]
        c = p2[:, :o] - p2[:, o:]
        var = jnp.mean(c * c, axis=1, keepdims=True)
        o_ref[...] = c * lax.rsqrt(var + EPS) * g_ref[...] + be_ref[...]

    wt_aug = jnp.concatenate(
        [wt, jnp.tile(jnp.mean(wt, axis=1, keepdims=True), (1, o))], axis=1
    )
    b_aug = jnp.concatenate(
        [b, jnp.full((1, o), jnp.mean(b), dtype=b.dtype)], axis=1
    )

    grid = (n_blk,)
    return pl.pallas_call(
        body,
        grid=grid,
        in_specs=[
            pl.BlockSpec((d, TBL_BLK), lambda i: (0, i)),
            pl.BlockSpec((d, 2 * o), lambda i: (0, 0)),
            pl.BlockSpec((1, 2 * o), lambda i: (0, 0)),
            pl.BlockSpec((1, o), lambda i: (0, 0)),
            pl.BlockSpec((1, o), lambda i: (0, 0)),
        ],
        out_specs=pl.BlockSpec((TBL_BLK, o), lambda i: (i, 0)),
        out_shape=jax.ShapeDtypeStruct((vp, o), jnp.float32),
        compiler_params=pltpu.CompilerParams(
            dimension_semantics=("arbitrary",),
        ),
    )(table_t, wt_aug, b_aug, gamma, beta)


def _sc_gather(ids3d, g_table, n, o):
    per_w = n // NW
    n_chunks = per_w // CH_IDX
    n_groups = n_chunks // NBUF
    mesh = plsc.VectorSubcoreMesh(
        core_axis_name="c", subcore_axis_name="s", num_cores=NC, num_subcores=NS
    )

    @functools.partial(
        pl.kernel,
        out_type=jax.ShapeDtypeStruct((n, o), jnp.float32),
        mesh=mesh,
        scratch_types=[
            pltpu.VMEM((n_chunks, CH_IDX), jnp.int32),
            pltpu.VMEM((NBUF, CH_IDX, o), jnp.float32),
            pltpu.SemaphoreType.DMA((NBUF,)),
            pltpu.SemaphoreType.DMA((NBUF,)),
        ],
        compiler_params=pltpu.CompilerParams(use_tc_tiling_on_sc=False),
    )
    def gather_kernel(ids_hbm, g_hbm, out_hbm, idx_v, rows_v, gsems, ssems):
        wid = lax.axis_index("s") * NC + lax.axis_index("c")
        base = wid * per_w
        pltpu.sync_copy(ids_hbm.at[wid], idx_v)

        def start_gather(j, buf):
            pltpu.async_copy(g_hbm.at[idx_v.at[j]], rows_v.at[buf], gsems.at[buf])

        def wait_gather(j, buf):
            pltpu.make_async_copy(
                g_hbm.at[idx_v.at[j]], rows_v.at[buf], gsems.at[buf]
            ).wait()

        def start_store(j, buf):
            pltpu.async_copy(
                rows_v.at[buf],
                out_hbm.at[pl.ds(base + j * CH_IDX, CH_IDX)],
                ssems.at[buf],
            )

        def wait_store(j, buf):
            pltpu.make_async_copy(
                rows_v.at[buf],
                out_hbm.at[pl.ds(base + j * CH_IDX, CH_IDX)],
                ssems.at[buf],
            ).wait()

        for buf in range(NBUF):
            start_gather(buf, buf)

        def group(g, carry):
            for buf in range(NBUF):
                j = g * NBUF + buf
                wait_gather(j, buf)
                start_store(j, buf)
            for buf in range(NBUF):
                j = g * NBUF + buf
                wait_store(j, buf)
                start_gather(j + NBUF, buf)
            return carry

        lax.fori_loop(0, n_groups - 1, group, 0, unroll=False)

        last = (n_groups - 1) * NBUF
        for buf in range(NBUF):
            wait_gather(last + buf, buf)
            start_store(last + buf, buf)
        for buf in range(NBUF):
            wait_store(last + buf, buf)

    return gather_kernel(ids3d, g_table)


@jax.jit
def kernel(item_ids, feat_matrix, W, b, gamma, beta):
    B, L = item_ids.shape
    v, d = feat_matrix.shape
    o = W.shape[0]
    n = B * L

    g_table = _tc_encode_table(
        feat_matrix.T,
        W.T,
        b.reshape(1, o),
        gamma.reshape(1, o),
        beta.reshape(1, o),
        v, d, o,
    )

    per_w = n // NW
    n_chunks = per_w // CH_IDX
    ids3d = item_ids.reshape(NW, n_chunks, CH_IDX)

    out = _sc_gather(ids3d, g_table, n, o)
    return out.reshape(B, L, o)
